# baseline (speedup 1.0000x reference)
import functools

import jax
import jax.numpy as jnp
from jax import lax
from jax.experimental import pallas as pl
from jax.experimental.pallas import tpu as pltpu
from jax.experimental.pallas import tpu_sc as plsc

NUM_FIELDS = 26
VOCAB = 100000
DIM = 8
BATCH = 16384

NC, NS = 2, 16
NW = NC * NS
B_PER_W = BATCH // NW
CHUNK = 128
QPF = B_PER_W // CHUNK

FH = NUM_FIELDS // 2
HCOL = FH * DIM
GROUP = 52
N_GATHERS = HCOL * QPF
N_GROUPS = N_GATHERS // GROUP


def _gather_body(xt_hbm, tt_hbm, out_hbm, idx_v, gbuf_v, sem, sem2):
    cid = lax.axis_index("c")
    sid = lax.axis_index("s")
    wid = sid * NC + cid

    def idx_fire(f, carry):
        pltpu.async_copy(xt_hbm.at[f, wid], idx_v.at[f], sem2)
        return carry

    lax.fori_loop(0, FH, idx_fire, 0, unroll=False)

    def idx_drain(f, carry):
        pltpu.make_async_copy(xt_hbm.at[f, wid], idx_v.at[f], sem2).wait()
        return carry

    lax.fori_loop(0, FH, idx_drain, 0, unroll=False)

    def gather_body(g, carry):
        descs = []
        for b in range(GROUP):
            j = g * GROUP + b
            e = j >> 2
            q = j & 3
            f = e >> 3
            descs.append(
                pltpu.async_copy(
                    tt_hbm.at[e].at[idx_v.at[f, q]],
                    gbuf_v.at[e, pl.ds(q * CHUNK, CHUNK)],
                    sem,
                )
            )
        for d in descs:
            d.wait()
        return carry

    lax.fori_loop(0, N_GROUPS, gather_body, 0, unroll=False)

    pltpu.sync_copy(gbuf_v, out_hbm.at[wid])


def _half_gather(xt_h, tt_h, mesh):
    return pl.kernel(
        _gather_body,
        out_type=jax.ShapeDtypeStruct((NW, HCOL, B_PER_W), jnp.float32),
        mesh=mesh,
        scratch_types=[
            pltpu.VMEM((FH, QPF, CHUNK), jnp.int32),
            pltpu.VMEM((HCOL, B_PER_W), jnp.float32),
            pltpu.SemaphoreType.DMA,
            pltpu.SemaphoreType.DMA,
        ],
        compiler_params=pltpu.CompilerParams(use_tc_tiling_on_sc=False),
    )(xt_h, tt_h)


@jax.jit
def _sc_embed(x, tables):
    mesh = plsc.VectorSubcoreMesh(
        core_axis_name="c", subcore_axis_name="s", num_cores=NC, num_subcores=NS
    )
    xt = x.astype(jnp.int32).T.reshape(NUM_FIELDS, NW, QPF, CHUNK)

    halves = []
    for h in range(2):
        tt_h = (
            tables[h * FH:(h + 1) * FH]
            .transpose(0, 2, 1)
            .reshape(HCOL, VOCAB)
        )
        halves.append(_half_gather(xt[h * FH:(h + 1) * FH], tt_h, mesh))

    out_cm = jnp.concatenate(halves, axis=1)
    return out_cm.transpose(0, 2, 1).reshape(BATCH, NUM_FIELDS * DIM)


def kernel(x, tables):
    return _sc_embed(x, tables)


# speedup vs baseline: 1.0633x; 1.0633x over previous
import jax
import jax.numpy as jnp
from jax import lax
from jax.experimental import pallas as pl
from jax.experimental.pallas import tpu as pltpu
from jax.experimental.pallas import tpu_sc as plsc

NUM_FIELDS = 26
VOCAB = 100000
DIM = 8
BATCH = 16384
NCOL = NUM_FIELDS * DIM

NC, NS = 2, 16
NW = NC * NS
B_PER_W = BATCH // NW
CHUNK = 128
QPF = B_PER_W // CHUNK
GROUP = 64
N_GATHERS = NCOL * QPF
N_GROUPS = N_GATHERS // GROUP


def _gather_body(xt_hbm, tt_hbm, out_hbm, idx_v, gbuf_v, sem, sem2):
    cid = lax.axis_index("c")
    sid = lax.axis_index("s")
    wid = sid * NC + cid

    def idx_fire(f, carry):
        pltpu.async_copy(xt_hbm.at[f, wid], idx_v.at[f], sem2)
        return carry

    lax.fori_loop(0, NUM_FIELDS, idx_fire, 0, unroll=False)

    def idx_drain(f, carry):
        pltpu.make_async_copy(xt_hbm.at[f, wid], idx_v.at[f], sem2).wait()
        return carry

    lax.fori_loop(0, NUM_FIELDS, idx_drain, 0, unroll=False)

    def gather_body(g, carry):
        descs = []
        for b in range(GROUP):
            j = g * GROUP + b
            e = j >> 2
            q = j & 3
            f = e >> 3
            descs.append(
                pltpu.async_copy(
                    tt_hbm.at[e].at[idx_v.at[f, q]],
                    gbuf_v.at[e, pl.ds(q * CHUNK, CHUNK)],
                    sem,
                )
            )
        for d in descs:
            d.wait()
        return carry

    lax.fori_loop(0, N_GROUPS, gather_body, 0, unroll=False)

    pltpu.sync_copy(gbuf_v, out_hbm.at[wid])


@jax.jit
def _sc_embed(x, tables):
    mesh = plsc.VectorSubcoreMesh(
        core_axis_name="c", subcore_axis_name="s", num_cores=NC, num_subcores=NS
    )
    tt = tables.transpose(0, 2, 1).reshape(NCOL, VOCAB)
    xt = x.astype(jnp.int32).T.reshape(NUM_FIELDS, NW, QPF, CHUNK)

    out_cm = pl.kernel(
        _gather_body,
        out_type=jax.ShapeDtypeStruct((NW, NCOL, B_PER_W), jnp.float32),
        mesh=mesh,
        scratch_types=[
            pltpu.VMEM((NUM_FIELDS, QPF, CHUNK), jnp.int32),
            pltpu.VMEM((NCOL, B_PER_W), jnp.float32),
            pltpu.SemaphoreType.DMA,
            pltpu.SemaphoreType.DMA,
        ],
        compiler_params=pltpu.CompilerParams(use_tc_tiling_on_sc=False),
    )(xt, tt)

    return out_cm.transpose(0, 2, 1).reshape(BATCH, NCOL)


def kernel(x, tables):
    return _sc_embed(x, tables)
